# baseline (speedup 1.0000x reference)
import functools

import jax
import jax.numpy as jnp
from jax import lax
from jax.experimental import pallas as pl
from jax.experimental.pallas import tpu as pltpu
from jax.experimental.pallas import tpu_sc as plsc


def _softmax_block_kernel(x_ref, lp_ref, p_ref):
    x = x_ref[...]
    m = jnp.max(x, axis=-1, keepdims=True)
    e = jnp.exp(x - m)
    s = jnp.sum(e, axis=-1, keepdims=True)
    lp_ref[...] = x - (m + jnp.log(s))
    p_ref[...] = e * (1.0 / s)


def _tc_softmax(xf):
    rows, N = xf.shape
    R = 64
    grid = (rows // R,)
    return pl.pallas_call(
        _softmax_block_kernel,
        grid=grid,
        in_specs=[pl.BlockSpec((R, N), lambda i: (i, 0))],
        out_specs=[
            pl.BlockSpec((R, N), lambda i: (i, 0)),
            pl.BlockSpec((R, N), lambda i: (i, 0)),
        ],
        out_shape=[
            jax.ShapeDtypeStruct((rows, N), xf.dtype),
            jax.ShapeDtypeStruct((rows, N), xf.dtype),
        ],
        compiler_params=pltpu.CompilerParams(
            dimension_semantics=("parallel",),
        ),
    )(xf)


_NC, _NS, _L = 2, 16, 16
_NW = _NC * _NS
_LN2 = 0.6931471805599453


def _vlog(sv):
    bits = lax.bitcast_convert_type(sv, jnp.int32)
    ev = (bits >> 23) - 127
    mant = lax.bitcast_convert_type(
        (bits & 0x7FFFFF) | 0x3F800000, jnp.float32)
    z = (mant - 1.0) / (mant + 1.0)
    z2 = z * z
    ln_m = 2.0 * z * (1.0 + z2 * (1.0 / 3.0 + z2 * (
        1.0 / 5.0 + z2 * (1.0 / 7.0 + z2 * (1.0 / 9.0)))))
    return ev.astype(jnp.float32) * _LN2 + ln_m


def _xlane_reduce(v, op):
    idx = lax.iota(jnp.int32, _L)
    for k in (1, 2, 4, 8):
        v = op(v, v.at[idx ^ k].get(mode="promise_in_bounds"))
    return v


def _sc_softmax_body(rows, N, unroll, x_hbm, lp_hbm, p_hbm,
                     xbuf0, xbuf1, pt0, pt1, pt2, pt3,
                     sin0, sin1, slp0, slp1, sp0, sp1, sp2, sp3):
    rpw = rows // _NW
    Q = N // 4
    wid = lax.axis_index("s") * _NC + lax.axis_index("c")
    base = wid * rpw
    xbufs, sins, slps = [xbuf0, xbuf1], [sin0, sin1], [slp0, slp1]
    pts, sps = [pt0, pt1, pt2, pt3], [sp0, sp1, sp2, sp3]
    in_h, lp_h, p_h = {}, {}, {}

    in_h[0] = pltpu.async_copy(x_hbm.at[base], xbufs[0], sins[0])

    for r in range(rpw):
        row = base + r
        buf = xbufs[r % 2]
        in_h[r].wait()
        if r + 1 < rpw:
            if r >= 1:
                lp_h[r - 1].wait()
            in_h[r + 1] = pltpu.async_copy(
                x_hbm.at[row + 1], xbufs[(r + 1) % 2], sins[(r + 1) % 2])

        def max_body(i, acc, buf=buf):
            for u in range(unroll):
                acc = jnp.maximum(acc, buf[pl.ds((i * unroll + u) * _L, _L)])
            return acc
        macc = lax.fori_loop(0, N // _L // unroll, max_body,
                             jnp.full((_L,), -jnp.inf, jnp.float32))
        m = _xlane_reduce(macc, jnp.maximum)

        def sum_body(i, acc, buf=buf, m=m):
            for u in range(unroll):
                v = buf[pl.ds((i * unroll + u) * _L, _L)]
                acc = acc + jnp.exp(v - m)
            return acc
        sacc = lax.fori_loop(0, N // _L // unroll, sum_body,
                             jnp.zeros((_L,), jnp.float32))
        lse = m + _vlog(_xlane_reduce(sacc, jnp.add))

        for q in range(4):
            if r > 0:
                p_h[(r - 1, q)].wait()

            def out_body(i, carry, buf=buf, pt=pts[q], q=q, lse=lse):
                for u in range(unroll):
                    k = i * unroll + u
                    lp = buf[pl.ds((q * (Q // _L) + k) * _L, _L)] - lse
                    pt[pl.ds(k * _L, _L)] = jnp.exp(lp)
                    buf[pl.ds((q * (Q // _L) + k) * _L, _L)] = lp
                return carry
            lax.fori_loop(0, Q // _L // unroll, out_body, 0)
            p_h[(r, q)] = pltpu.async_copy(
                pts[q], p_hbm.at[row, pl.ds(q * Q, Q)], sps[q])
        lp_h[r] = pltpu.async_copy(buf, lp_hbm.at[row], slps[r % 2])

    lp_h[rpw - 2].wait()
    lp_h[rpw - 1].wait()
    for q in range(4):
        p_h[(rpw - 1, q)].wait()


def _sc_softmax(xf, unroll=8):
    rows, N = xf.shape
    mesh = plsc.VectorSubcoreMesh(core_axis_name="c", subcore_axis_name="s")
    body = functools.partial(_sc_softmax_body, rows, N, unroll)
    return pl.kernel(
        body,
        out_type=[
            jax.ShapeDtypeStruct((rows, N), jnp.float32),
            jax.ShapeDtypeStruct((rows, N), jnp.float32),
        ],
        mesh=mesh,
        scratch_types=[
            pltpu.VMEM((N,), jnp.float32),
            pltpu.VMEM((N,), jnp.float32),
            pltpu.VMEM((N // 4,), jnp.float32),
            pltpu.VMEM((N // 4,), jnp.float32),
            pltpu.VMEM((N // 4,), jnp.float32),
            pltpu.VMEM((N // 4,), jnp.float32),
            pltpu.SemaphoreType.DMA,
            pltpu.SemaphoreType.DMA,
            pltpu.SemaphoreType.DMA,
            pltpu.SemaphoreType.DMA,
            pltpu.SemaphoreType.DMA,
            pltpu.SemaphoreType.DMA,
            pltpu.SemaphoreType.DMA,
            pltpu.SemaphoreType.DMA,
        ],
    )(xf)


def _copy_block_kernel(x_ref, lp_ref, p_ref):
    x = x_ref[...]
    lp_ref[...] = x
    p_ref[...] = x


def _tc_copy_probe(xf):
    rows, N = xf.shape
    R = 64
    grid = (rows // R,)
    return pl.pallas_call(
        _copy_block_kernel,
        grid=grid,
        in_specs=[pl.BlockSpec((R, N), lambda i: (i, 0))],
        out_specs=[
            pl.BlockSpec((R, N), lambda i: (i, 0)),
            pl.BlockSpec((R, N), lambda i: (i, 0)),
        ],
        out_shape=[
            jax.ShapeDtypeStruct((rows, N), xf.dtype),
            jax.ShapeDtypeStruct((rows, N), xf.dtype),
        ],
        compiler_params=pltpu.CompilerParams(
            dimension_semantics=("parallel",),
        ),
    )(xf)


def kernel(x):
    B, H, N = x.shape
    xf = x.reshape(B * H, N)
    lp, p = _sc_softmax(xf)
    return lp.reshape(B, H, N), p.reshape(B, H, N)


# speedup vs baseline: 1.1039x; 1.1039x over previous
import functools

import jax
import jax.numpy as jnp
from jax import lax
from jax.experimental import pallas as pl
from jax.experimental.pallas import tpu as pltpu
from jax.experimental.pallas import tpu_sc as plsc


def _softmax_block_kernel(x_ref, lp_ref, p_ref):
    x = x_ref[...]
    m = jnp.max(x, axis=-1, keepdims=True)
    e = jnp.exp(x - m)
    s = jnp.sum(e, axis=-1, keepdims=True)
    lp_ref[...] = x - (m + jnp.log(s))
    p_ref[...] = e * (1.0 / s)


def _tc_softmax(xf):
    rows, N = xf.shape
    R = 64
    grid = (rows // R,)
    return pl.pallas_call(
        _softmax_block_kernel,
        grid=grid,
        in_specs=[pl.BlockSpec((R, N), lambda i: (i, 0))],
        out_specs=[
            pl.BlockSpec((R, N), lambda i: (i, 0)),
            pl.BlockSpec((R, N), lambda i: (i, 0)),
        ],
        out_shape=[
            jax.ShapeDtypeStruct((rows, N), xf.dtype),
            jax.ShapeDtypeStruct((rows, N), xf.dtype),
        ],
        compiler_params=pltpu.CompilerParams(
            dimension_semantics=("parallel",),
        ),
    )(xf)


_NC, _NS, _L = 2, 16, 16
_NW = _NC * _NS
_LN2 = 0.6931471805599453


def _vlog(sv):
    bits = lax.bitcast_convert_type(sv, jnp.int32)
    ev = (bits >> 23) - 127
    mant = lax.bitcast_convert_type(
        (bits & 0x7FFFFF) | 0x3F800000, jnp.float32)
    z = (mant - 1.0) / (mant + 1.0)
    z2 = z * z
    ln_m = 2.0 * z * (1.0 + z2 * (1.0 / 3.0 + z2 * (
        1.0 / 5.0 + z2 * (1.0 / 7.0 + z2 * (1.0 / 9.0)))))
    return ev.astype(jnp.float32) * _LN2 + ln_m


def _xlane_reduce(v, op):
    idx = lax.iota(jnp.int32, _L)
    for k in (1, 2, 4, 8):
        v = op(v, v.at[idx ^ k].get(mode="promise_in_bounds"))
    return v


def _sc_softmax_body(rows, N, unroll, x_hbm, lp_hbm, p_hbm,
                     xbuf0, xbuf1, pt0, pt1, pt2, pt3,
                     sin0, sin1, slp0, slp1, sp0, sp1, sp2, sp3):
    rpw = rows // _NW
    Q = N // 4
    wid = lax.axis_index("s") * _NC + lax.axis_index("c")
    base = wid * rpw
    xbufs, sins, slps = [xbuf0, xbuf1], [sin0, sin1], [slp0, slp1]
    pts, sps = [pt0, pt1, pt2, pt3], [sp0, sp1, sp2, sp3]
    in_h, lp_h, p_h = {}, {}, {}

    in_h[0] = pltpu.async_copy(x_hbm.at[base], xbufs[0], sins[0])

    for r in range(rpw):
        row = base + r
        buf = xbufs[r % 2]
        in_h[r].wait()
        if r + 1 < rpw:
            if r >= 1:
                lp_h[r - 1].wait()
            in_h[r + 1] = pltpu.async_copy(
                x_hbm.at[row + 1], xbufs[(r + 1) % 2], sins[(r + 1) % 2])

        nacc = 4

        def max_body(i, accs, buf=buf):
            accs = list(accs)
            for u in range(unroll):
                accs[u % nacc] = jnp.maximum(
                    accs[u % nacc], buf[pl.ds((i * unroll + u) * _L, _L)])
            return tuple(accs)
        maccs = lax.fori_loop(
            0, N // _L // unroll, max_body,
            tuple(jnp.full((_L,), -jnp.inf, jnp.float32) for _ in range(nacc)))
        m = _xlane_reduce(functools.reduce(jnp.maximum, maccs), jnp.maximum)

        def sum_body(i, accs, buf=buf, m=m):
            accs = list(accs)
            for u in range(unroll):
                v = buf[pl.ds((i * unroll + u) * _L, _L)]
                accs[u % nacc] = accs[u % nacc] + jnp.exp(v - m)
            return tuple(accs)
        saccs = lax.fori_loop(
            0, N // _L // unroll, sum_body,
            tuple(jnp.zeros((_L,), jnp.float32) for _ in range(nacc)))
        lse = m + _vlog(_xlane_reduce(functools.reduce(jnp.add, saccs),
                                      jnp.add))

        for q in range(4):
            if r > 0:
                p_h[(r - 1, q)].wait()

            def out_body(i, carry, buf=buf, pt=pts[q], q=q, lse=lse):
                for u in range(unroll):
                    k = i * unroll + u
                    lp = buf[pl.ds((q * (Q // _L) + k) * _L, _L)] - lse
                    pt[pl.ds(k * _L, _L)] = jnp.exp(lp)
                    buf[pl.ds((q * (Q // _L) + k) * _L, _L)] = lp
                return carry
            lax.fori_loop(0, Q // _L // unroll, out_body, 0)
            p_h[(r, q)] = pltpu.async_copy(
                pts[q], p_hbm.at[row, pl.ds(q * Q, Q)], sps[q])
        lp_h[r] = pltpu.async_copy(buf, lp_hbm.at[row], slps[r % 2])

    lp_h[rpw - 2].wait()
    lp_h[rpw - 1].wait()
    for q in range(4):
        p_h[(rpw - 1, q)].wait()


def _sc_softmax(xf, unroll=8):
    rows, N = xf.shape
    mesh = plsc.VectorSubcoreMesh(core_axis_name="c", subcore_axis_name="s")
    body = functools.partial(_sc_softmax_body, rows, N, unroll)
    return pl.kernel(
        body,
        out_type=[
            jax.ShapeDtypeStruct((rows, N), jnp.float32),
            jax.ShapeDtypeStruct((rows, N), jnp.float32),
        ],
        mesh=mesh,
        scratch_types=[
            pltpu.VMEM((N,), jnp.float32),
            pltpu.VMEM((N,), jnp.float32),
            pltpu.VMEM((N // 4,), jnp.float32),
            pltpu.VMEM((N // 4,), jnp.float32),
            pltpu.VMEM((N // 4,), jnp.float32),
            pltpu.VMEM((N // 4,), jnp.float32),
            pltpu.SemaphoreType.DMA,
            pltpu.SemaphoreType.DMA,
            pltpu.SemaphoreType.DMA,
            pltpu.SemaphoreType.DMA,
            pltpu.SemaphoreType.DMA,
            pltpu.SemaphoreType.DMA,
            pltpu.SemaphoreType.DMA,
            pltpu.SemaphoreType.DMA,
        ],
    )(xf)


def _copy_block_kernel(x_ref, lp_ref, p_ref):
    x = x_ref[...]
    lp_ref[...] = x
    p_ref[...] = x


def _tc_copy_probe(xf):
    rows, N = xf.shape
    R = 64
    grid = (rows // R,)
    return pl.pallas_call(
        _copy_block_kernel,
        grid=grid,
        in_specs=[pl.BlockSpec((R, N), lambda i: (i, 0))],
        out_specs=[
            pl.BlockSpec((R, N), lambda i: (i, 0)),
            pl.BlockSpec((R, N), lambda i: (i, 0)),
        ],
        out_shape=[
            jax.ShapeDtypeStruct((rows, N), xf.dtype),
            jax.ShapeDtypeStruct((rows, N), xf.dtype),
        ],
        compiler_params=pltpu.CompilerParams(
            dimension_semantics=("parallel",),
        ),
    )(xf)


def kernel(x):
    B, H, N = x.shape
    xf = x.reshape(B * H, N)
    lp, p = _sc_softmax(xf)
    return lp.reshape(B, H, N), p.reshape(B, H, N)


# speedup vs baseline: 2.4688x; 2.2365x over previous
import functools

import jax
import jax.numpy as jnp
from jax import lax
from jax.experimental import pallas as pl
from jax.experimental.pallas import tpu as pltpu
from jax.experimental.pallas import tpu_sc as plsc


def _softmax_block_kernel(x_ref, lp_ref, p_ref):
    x = x_ref[...]
    m = jnp.max(x, axis=-1, keepdims=True)
    e = jnp.exp(x - m)
    s = jnp.sum(e, axis=-1, keepdims=True)
    lp_ref[...] = x - (m + jnp.log(s))
    p_ref[...] = e * (1.0 / s)


def _tc_softmax(xf):
    rows, N = xf.shape
    R = 64
    grid = (rows // R,)
    return pl.pallas_call(
        _softmax_block_kernel,
        grid=grid,
        in_specs=[pl.BlockSpec((R, N), lambda i: (i, 0))],
        out_specs=[
            pl.BlockSpec((R, N), lambda i: (i, 0)),
            pl.BlockSpec((R, N), lambda i: (i, 0)),
        ],
        out_shape=[
            jax.ShapeDtypeStruct((rows, N), xf.dtype),
            jax.ShapeDtypeStruct((rows, N), xf.dtype),
        ],
        compiler_params=pltpu.CompilerParams(
            dimension_semantics=("parallel",),
        ),
    )(xf)


_NC, _NS, _L = 2, 16, 16
_NW = _NC * _NS
_LN2 = 0.6931471805599453


def _vlog(sv):
    bits = lax.bitcast_convert_type(sv, jnp.int32)
    ev = (bits >> 23) - 127
    mant = lax.bitcast_convert_type(
        (bits & 0x7FFFFF) | 0x3F800000, jnp.float32)
    z = (mant - 1.0) / (mant + 1.0)
    z2 = z * z
    ln_m = 2.0 * z * (1.0 + z2 * (1.0 / 3.0 + z2 * (
        1.0 / 5.0 + z2 * (1.0 / 7.0 + z2 * (1.0 / 9.0)))))
    return ev.astype(jnp.float32) * _LN2 + ln_m


def _xlane_reduce(v, op):
    idx = lax.iota(jnp.int32, _L)
    for k in (1, 2, 4, 8):
        v = op(v, v.at[idx ^ k].get(mode="promise_in_bounds"))
    return v


def _sc_softmax_body(rows, N, unroll, x_hbm, lp_hbm, p_hbm,
                     xbuf0, xbuf1, pt0, pt1, pt2, pt3,
                     sin0, sin1, slp0, slp1, sp0, sp1, sp2, sp3):
    rpw = rows // _NW
    Q = N // 4
    wid = lax.axis_index("s") * _NC + lax.axis_index("c")
    base = wid * rpw
    xbufs, sins, slps = [xbuf0, xbuf1], [sin0, sin1], [slp0, slp1]
    pts, sps = [pt0, pt1, pt2, pt3], [sp0, sp1, sp2, sp3]
    in_h, lp_h, p_h = {}, {}, {}

    in_h[0] = pltpu.async_copy(x_hbm.at[base], xbufs[0], sins[0])

    for r in range(rpw):
        row = base + r
        buf = xbufs[r % 2]
        in_h[r].wait()
        if r + 1 < rpw:
            if r >= 1:
                lp_h[r - 1].wait()
            in_h[r + 1] = pltpu.async_copy(
                x_hbm.at[row + 1], xbufs[(r + 1) % 2], sins[(r + 1) % 2])

        nacc = 4

        def max_body(i, accs, buf=buf):
            accs = list(accs)
            for u in range(unroll):
                accs[u % nacc] = jnp.maximum(
                    accs[u % nacc], buf[pl.ds((i * unroll + u) * _L, _L)])
            return tuple(accs)
        maccs = lax.fori_loop(
            0, N // _L // unroll, max_body,
            tuple(jnp.full((_L,), -jnp.inf, jnp.float32) for _ in range(nacc)))
        m = _xlane_reduce(functools.reduce(jnp.maximum, maccs), jnp.maximum)

        def sum_body(i, accs, buf=buf, m=m):
            accs = list(accs)
            for u in range(unroll):
                v = buf[pl.ds((i * unroll + u) * _L, _L)]
                accs[u % nacc] = accs[u % nacc] + jnp.exp(v - m)
            return tuple(accs)
        saccs = lax.fori_loop(
            0, N // _L // unroll, sum_body,
            tuple(jnp.zeros((_L,), jnp.float32) for _ in range(nacc)))
        lse = m + _vlog(_xlane_reduce(functools.reduce(jnp.add, saccs),
                                      jnp.add))

        for q in range(4):
            if r > 0:
                p_h[(r - 1, q)].wait()

            def out_body(i, carry, buf=buf, pt=pts[q], q=q, lse=lse):
                for u in range(unroll):
                    k = i * unroll + u
                    lp = buf[pl.ds((q * (Q // _L) + k) * _L, _L)] - lse
                    pt[pl.ds(k * _L, _L)] = jnp.exp(lp)
                    buf[pl.ds((q * (Q // _L) + k) * _L, _L)] = lp
                return carry
            lax.fori_loop(0, Q // _L // unroll, out_body, 0)
            p_h[(r, q)] = pltpu.async_copy(
                pts[q], p_hbm.at[row, pl.ds(q * Q, Q)], sps[q])
        lp_h[r] = pltpu.async_copy(buf, lp_hbm.at[row], slps[r % 2])

    lp_h[rpw - 2].wait()
    lp_h[rpw - 1].wait()
    for q in range(4):
        p_h[(rpw - 1, q)].wait()


def _sc_softmax(xf, unroll=8):
    rows, N = xf.shape
    mesh = plsc.VectorSubcoreMesh(core_axis_name="c", subcore_axis_name="s")
    body = functools.partial(_sc_softmax_body, rows, N, unroll)
    return pl.kernel(
        body,
        out_type=[
            jax.ShapeDtypeStruct((rows, N), jnp.float32),
            jax.ShapeDtypeStruct((rows, N), jnp.float32),
        ],
        mesh=mesh,
        scratch_types=[
            pltpu.VMEM((N,), jnp.float32),
            pltpu.VMEM((N,), jnp.float32),
            pltpu.VMEM((N // 4,), jnp.float32),
            pltpu.VMEM((N // 4,), jnp.float32),
            pltpu.VMEM((N // 4,), jnp.float32),
            pltpu.VMEM((N // 4,), jnp.float32),
            pltpu.SemaphoreType.DMA,
            pltpu.SemaphoreType.DMA,
            pltpu.SemaphoreType.DMA,
            pltpu.SemaphoreType.DMA,
            pltpu.SemaphoreType.DMA,
            pltpu.SemaphoreType.DMA,
            pltpu.SemaphoreType.DMA,
            pltpu.SemaphoreType.DMA,
        ],
    )(xf)


def kernel(x):
    B, H, N = x.shape
    xf = x.reshape(B * H, N)
    lp, p = _tc_softmax(xf)
    return lp.reshape(B, H, N), p.reshape(B, H, N)
